# baseline (speedup 1.0000x reference)
import functools

import jax
import jax.numpy as jnp
from jax import lax
from jax.experimental import pallas as pl
from jax.experimental.pallas import tpu as pltpu
from jax.experimental.pallas import tpu_sc as plsc

NUM_CORES = 2
NUM_SUBCORES = 16
NUM_WORKERS = NUM_CORES * NUM_SUBCORES
LANES = 16


@functools.partial(jax.jit, static_argnums=(3, 4, 5))
def _embed_sc(idx, tok_table, pos_table, batch, seq, chunk):
    d_model = tok_table.shape[1]
    rows = batch * seq
    spw = seq // NUM_WORKERS
    n_sc = spw // chunk
    pieces = d_model // LANES
    grows = batch * chunk

    mesh = plsc.VectorSubcoreMesh(
        core_axis_name="c", subcore_axis_name="s",
        num_cores=NUM_CORES, num_subcores=NUM_SUBCORES,
    )

    @functools.partial(
        pl.kernel,
        mesh=mesh,
        out_type=jax.ShapeDtypeStruct((rows, d_model), jnp.float32),
        scratch_types=[
            pltpu.VMEM((n_sc, grows), jnp.int32),

            pltpu.VMEM((chunk, d_model), jnp.float32),
            pltpu.VMEM((chunk, d_model), jnp.float32),
            pltpu.VMEM((grows, d_model), jnp.float32),
            pltpu.VMEM((grows, d_model), jnp.float32),
            pltpu.SemaphoreType.DMA,
            pltpu.SemaphoreType.DMA,
            pltpu.SemaphoreType.DMA,
            pltpu.SemaphoreType.DMA,
            pltpu.SemaphoreType.DMA,
            pltpu.SemaphoreType.DMA,
        ],
    )
    def body(idx_hbm, tok_hbm, pos_hbm, out_hbm,
             idx_v, pbuf0, pbuf1, tbuf0, tbuf1,
             gs0, gs1, os0, os1, ps0, ps1):
        tb = (tbuf0, tbuf1)
        pb = (pbuf0, pbuf1)
        gs = (gs0, gs1)
        osem = (os0, os1)
        psem = (ps0, ps1)

        wid = lax.axis_index("s") * NUM_CORES + lax.axis_index("c")
        s_base = wid * spw

        pltpu.sync_copy(idx_hbm.at[wid], idx_v)

        def pos_issue(sc, k):
            pltpu.async_copy(pos_hbm.at[pl.ds(s_base + sc * chunk, chunk)],
                             pb[k], psem[k])

        def pos_wait(k):
            pltpu.make_async_copy(pos_hbm.at[pl.ds(0, chunk)], pb[k],
                                  psem[k]).wait()

        def gather_issue(sc, k):
            pltpu.async_copy(tok_hbm.at[idx_v.at[sc]], tb[k], gs[k])

        def gather_wait(sc, k):
            pltpu.make_async_copy(tok_hbm.at[idx_v.at[sc]], tb[k],
                                  gs[k]).wait()

        def outs_drain(k):
            for b in range(batch):
                pltpu.make_async_copy(
                    tb[k].at[pl.ds(b * chunk, chunk)],
                    out_hbm.at[pl.ds(0, chunk)], osem[k]).wait()

        pos_issue(0, 0)
        gather_issue(0, 0)

        def outer(sc, _):
            kp = lax.rem(sc, 2)
            for k in range(2):
                @pl.when(kp == k)
                def _():
                    nk = 1 - k

                    @pl.when(sc + 1 < n_sc)
                    def _():
                        pos_issue(sc + 1, nk)

                        @pl.when(sc > 0)
                        def _():
                            outs_drain(nk)

                        gather_issue(sc + 1, nk)

                    pos_wait(k)
                    gather_wait(sc, k)

                    def add_row(r, _):
                        for j in range(pieces):
                            sl = pl.ds(j * LANES, LANES)
                            p = pb[k][r, sl]
                            for b in range(batch):
                                tb[k][b * chunk + r, sl] = (
                                    tb[k][b * chunk + r, sl] + p)
                        return 0

                    lax.fori_loop(0, chunk, add_row, 0)
                    for b in range(batch):
                        pltpu.async_copy(
                            tb[k].at[pl.ds(b * chunk, chunk)],
                            out_hbm.at[
                                pl.ds(b * seq + s_base + sc * chunk, chunk)],
                            osem[k])
            return 0

        lax.fori_loop(0, n_sc, outer, 0)
        outs_drain((n_sc - 2) % 2)
        outs_drain((n_sc - 1) % 2)

    return body(idx, tok_table, pos_table)


def kernel(x, tok_table, pos_table):
    batch, seq = x.shape
    d_model = tok_table.shape[1]
    chunk = 16
    spw = seq // NUM_WORKERS
    n_sc = spw // chunk
    idx = (x.astype(jnp.int32)
           .reshape(batch, NUM_WORKERS, n_sc, chunk)
           .transpose(1, 2, 0, 3)
           .reshape(NUM_WORKERS, n_sc, batch * chunk))
    out = _embed_sc(idx, tok_table, pos_table, batch, seq, chunk)
    return out.reshape(batch, seq, d_model)


# speedup vs baseline: 1.9399x; 1.9399x over previous
import functools

import jax
import jax.numpy as jnp
from jax import lax
from jax.experimental import pallas as pl
from jax.experimental.pallas import tpu as pltpu
from jax.experimental.pallas import tpu_sc as plsc

NUM_CORES = 2
NUM_SUBCORES = 16
NUM_WORKERS = NUM_CORES * NUM_SUBCORES
LANES = 16


@functools.partial(jax.jit, static_argnums=(3, 4, 5))
def _embed_sc(idx, tok_table, pos_table, batch, seq, chunk):
    d_model = tok_table.shape[1]
    rows = batch * seq
    spw = seq // NUM_WORKERS
    n_sc = spw // chunk
    pieces = d_model // LANES

    mesh = plsc.VectorSubcoreMesh(
        core_axis_name="c", subcore_axis_name="s",
        num_cores=NUM_CORES, num_subcores=NUM_SUBCORES,
    )

    tok_bufs = [pltpu.VMEM((chunk, d_model), jnp.float32)
                for _ in range(2 * batch)]

    @functools.partial(
        pl.kernel,
        mesh=mesh,
        out_type=jax.ShapeDtypeStruct((rows, d_model), jnp.float32),
        scratch_types=[
            pltpu.VMEM((batch, spw), jnp.int32),
            pltpu.VMEM((chunk, d_model), jnp.float32),
            pltpu.VMEM((chunk, d_model), jnp.float32),
            *tok_bufs,
            pltpu.SemaphoreType.DMA,
            pltpu.SemaphoreType.DMA,
            pltpu.SemaphoreType.DMA,
            pltpu.SemaphoreType.DMA,
            pltpu.SemaphoreType.DMA,
            pltpu.SemaphoreType.DMA,
        ],
    )
    def body(idx_hbm, tok_hbm, pos_hbm, out_hbm,
             idx_v, pbuf0, pbuf1, *rest):
        tbufs = rest[:2 * batch]
        gs0, gs1, os0, os1, ps0, ps1 = rest[2 * batch:]
        tb = (tbufs[:batch], tbufs[batch:])
        pb = (pbuf0, pbuf1)
        gs = (gs0, gs1)
        osem = (os0, os1)
        psem = (ps0, ps1)

        wid = lax.axis_index("s") * NUM_CORES + lax.axis_index("c")
        s_base = wid * spw

        for b in range(batch):
            pltpu.sync_copy(idx_hbm.at[pl.ds(b * seq + s_base, spw)],
                            idx_v.at[b])

        def pos_issue(sc, k):
            pltpu.async_copy(pos_hbm.at[pl.ds(s_base + sc * chunk, chunk)],
                             pb[k], psem[k])

        def pos_wait(k):
            pltpu.make_async_copy(pos_hbm.at[pl.ds(0, chunk)], pb[k],
                                  psem[k]).wait()

        def gathers_issue(sc, k):
            for b in range(batch):
                pltpu.async_copy(
                    tok_hbm.at[idx_v.at[b, pl.ds(sc * chunk, chunk)]],
                    tb[k][b], gs[k])

        def gathers_wait(sc, k):
            for b in range(batch):
                pltpu.make_async_copy(
                    tok_hbm.at[idx_v.at[b, pl.ds(sc * chunk, chunk)]],
                    tb[k][b], gs[k]).wait()

        def outs_drain(k):
            for b in range(batch):
                pltpu.make_async_copy(
                    tb[k][b], out_hbm.at[pl.ds(0, chunk)], osem[k]).wait()

        pos_issue(0, 0)
        gathers_issue(0, 0)

        def outer(sc, _):
            kp = lax.rem(sc, 2)
            for k in range(2):
                @pl.when(kp == k)
                def _():
                    nk = 1 - k

                    @pl.when(sc + 1 < n_sc)
                    def _():
                        pos_issue(sc + 1, nk)

                        @pl.when(sc > 0)
                        def _():
                            outs_drain(nk)

                        gathers_issue(sc + 1, nk)

                    pos_wait(k)
                    gathers_wait(sc, k)

                    def add_row(r, _):
                        for j in range(pieces):
                            sl = pl.ds(j * LANES, LANES)
                            p = pb[k][r, sl]
                            for b in range(batch):
                                plsc.addupdate(tb[k][b].at[r, sl], p)
                        return 0

                    lax.fori_loop(0, chunk, add_row, 0)
                    for b in range(batch):
                        pltpu.async_copy(
                            tb[k][b],
                            out_hbm.at[
                                pl.ds(b * seq + s_base + sc * chunk, chunk)],
                            osem[k])
            return 0

        lax.fori_loop(0, n_sc, outer, 0)
        outs_drain((n_sc - 2) % 2)
        outs_drain((n_sc - 1) % 2)

    return body(idx, tok_table, pos_table)


def kernel(x, tok_table, pos_table):
    batch, seq = x.shape
    d_model = tok_table.shape[1]
    idx = x.reshape(-1).astype(jnp.int32)
    out = _embed_sc(idx, tok_table, pos_table, batch, seq, 16)
    return out.reshape(batch, seq, d_model)


# speedup vs baseline: 2.0004x; 1.0312x over previous
import functools

import jax
import jax.numpy as jnp
from jax import lax
from jax.experimental import pallas as pl
from jax.experimental.pallas import tpu as pltpu
from jax.experimental.pallas import tpu_sc as plsc

NUM_CORES = 2
NUM_SUBCORES = 16
NUM_WORKERS = NUM_CORES * NUM_SUBCORES
LANES = 16


@functools.partial(jax.jit, static_argnums=(3, 4, 5))
def _embed_sc(idx, tok_table, pos_table, batch, seq, chunk):
    d_model = tok_table.shape[1]
    rows = batch * seq
    spw = seq // NUM_WORKERS
    n_sc = spw // chunk
    pieces = d_model // LANES

    mesh = plsc.VectorSubcoreMesh(
        core_axis_name="c", subcore_axis_name="s",
        num_cores=NUM_CORES, num_subcores=NUM_SUBCORES,
    )

    tok_bufs = [pltpu.VMEM((chunk, d_model), jnp.float32)
                for _ in range(2 * batch)]

    @functools.partial(
        pl.kernel,
        mesh=mesh,
        out_type=jax.ShapeDtypeStruct((rows, d_model), jnp.float32),
        scratch_types=[
            pltpu.VMEM((batch, spw), jnp.int32),
            pltpu.VMEM((chunk, d_model), jnp.float32),
            pltpu.VMEM((chunk, d_model), jnp.float32),
            *tok_bufs,
            pltpu.SemaphoreType.DMA,
            pltpu.SemaphoreType.DMA,
            pltpu.SemaphoreType.DMA,
            pltpu.SemaphoreType.DMA,
            pltpu.SemaphoreType.DMA,
            pltpu.SemaphoreType.DMA,
        ],
    )
    def body(idx_hbm, tok_hbm, pos_hbm, out_hbm,
             idx_v, pbuf0, pbuf1, *rest):
        tbufs = rest[:2 * batch]
        gs0, gs1, os0, os1, ps0, ps1 = rest[2 * batch:]
        tb = (tbufs[:batch], tbufs[batch:])
        pb = (pbuf0, pbuf1)
        gs = (gs0, gs1)
        osem = (os0, os1)
        psem = (ps0, ps1)

        wid = lax.axis_index("s") * NUM_CORES + lax.axis_index("c")
        s_base = wid * spw

        for b in range(batch):
            pltpu.sync_copy(idx_hbm.at[pl.ds(b * seq + s_base, spw)],
                            idx_v.at[b])

        def pos_issue(sc, k):
            pltpu.async_copy(pos_hbm.at[pl.ds(s_base + sc * chunk, chunk)],
                             pb[k], psem[k])

        def pos_wait(k):
            pltpu.make_async_copy(pos_hbm.at[pl.ds(0, chunk)], pb[k],
                                  psem[k]).wait()

        def gathers_issue(sc, k):
            for b in range(batch):
                pltpu.async_copy(
                    tok_hbm.at[idx_v.at[b, pl.ds(sc * chunk, chunk)]],
                    tb[k][b], gs[k])

        def gathers_wait(sc, k):
            for b in range(batch):
                pltpu.make_async_copy(
                    tok_hbm.at[idx_v.at[b, pl.ds(sc * chunk, chunk)]],
                    tb[k][b], gs[k]).wait()

        def outs_drain(k):
            for b in range(batch):
                pltpu.make_async_copy(
                    tb[k][b], out_hbm.at[pl.ds(0, chunk)], osem[k]).wait()

        pos_issue(0, 0)
        gathers_issue(0, 0)

        def outer(sc, _):
            kp = lax.rem(sc, 2)
            for k in range(2):
                @pl.when(kp == k)
                def _():
                    nk = 1 - k

                    @pl.when(sc + 1 < n_sc)
                    def _():
                        pos_issue(sc + 1, nk)

                        @pl.when(sc > 0)
                        def _():
                            outs_drain(nk)

                        gathers_issue(sc + 1, nk)

                    pos_wait(k)
                    gathers_wait(sc, k)

                    def add_row(r, _):
                        for j in range(pieces):
                            sl = pl.ds(j * LANES, LANES)
                            p = pb[k][r, sl]
                            for b in range(batch):
                                plsc.addupdate(tb[k][b].at[r, sl], p)
                        return 0

                    lax.fori_loop(0, chunk, add_row, 0)
                    for b in range(batch):
                        pltpu.async_copy(
                            tb[k][b],
                            out_hbm.at[
                                pl.ds(b * seq + s_base + sc * chunk, chunk)],
                            osem[k])
            return 0

        lax.fori_loop(0, n_sc, outer, 0)
        outs_drain((n_sc - 2) % 2)
        outs_drain((n_sc - 1) % 2)

    return body(idx, tok_table, pos_table)


def kernel(x, tok_table, pos_table):
    batch, seq = x.shape
    d_model = tok_table.shape[1]
    idx = x.reshape(-1).astype(jnp.int32)
    out = _embed_sc(idx, tok_table, pos_table, batch, seq, 8)
    return out.reshape(batch, seq, d_model)
